# Initial kernel scaffold; baseline (speedup 1.0000x reference)
import jax
import jax.numpy as jnp
from jax.experimental import pallas as pl


def kernel(graph_nodes, spx_image):
    raise NotImplementedError("write your pallas kernel here")


# speedup vs baseline: 2.0421x; 2.0421x over previous
import functools

import jax
import jax.numpy as jnp
from jax import lax
from jax.experimental import pallas as pl
from jax.experimental.pallas import tpu as pltpu
from jax.experimental.pallas import tpu_sc as plsc

_V = 10000
_VPAD = 10240
_D = 128
_H = 512
_W = 512
_P = _H * _W

_NC = 2
_NS = 16
_NW = _NC * _NS
_CPW = _D // _NW
_CHUNK = 8192
_NCHUNK = _P // _CHUNK
_LANES = 16


def _transpose_body(x_ref, o_ref):
    o_ref[...] = x_ref[...].T


def _table_transpose(table_pad):
    return pl.pallas_call(
        _transpose_body,
        out_shape=jax.ShapeDtypeStruct((_D, _VPAD), jnp.float32),
    )(table_pad)


def _gather_body(table_t_hbm, idx_hbm, out_hbm, rows_v, idx_v, out_v):
    wid = lax.axis_index("s") * _NC + lax.axis_index("c")
    c0 = wid * _CPW
    pltpu.sync_copy(table_t_hbm.at[pl.ds(c0 * _VPAD, _CPW * _VPAD)], rows_v)

    def chunk_body(g, carry):
        base = g * _CHUNK
        pltpu.sync_copy(idx_hbm.at[pl.ds(base, _CHUNK)], idx_v)

        def vec_body(i, carry2):
            p = i * _LANES
            idx16 = idx_v[pl.ds(p, _LANES)]
            for c in range(_CPW):
                out_v[pl.ds(c * _CHUNK + p, _LANES)] = plsc.load_gather(
                    rows_v, [idx16 + (c * _VPAD)])
            return carry2

        lax.fori_loop(0, _CHUNK // _LANES, vec_body, 0)
        for c in range(_CPW):
            pltpu.sync_copy(out_v.at[pl.ds(c * _CHUNK, _CHUNK)],
                            out_hbm.at[c0 + c, pl.ds(base, _CHUNK)])
        return carry

    lax.fori_loop(0, _NCHUNK, chunk_body, 0)


_gather_call = functools.partial(
    pl.kernel,
    out_type=jax.ShapeDtypeStruct((_D, _P), jnp.float32),
    mesh=plsc.VectorSubcoreMesh(core_axis_name="c", subcore_axis_name="s"),
    scratch_types=[
        pltpu.VMEM((_CPW * _VPAD,), jnp.float32),
        pltpu.VMEM((_CHUNK,), jnp.int32),
        pltpu.VMEM((_CPW * _CHUNK,), jnp.float32),
    ],
    compiler_params=pltpu.CompilerParams(needs_layout_passes=False),
)(_gather_body)


def kernel(graph_nodes, spx_image):
    table_pad = jnp.pad(graph_nodes, ((0, _VPAD - _V), (0, 0)))
    table_t = _table_transpose(table_pad).reshape(_D * _VPAD)
    idx = spx_image.reshape(-1).astype(jnp.int32)
    out = _gather_call(table_t, idx)
    return out.reshape(_D, _H, _W)


# speedup vs baseline: 4.8175x; 2.3591x over previous
import functools

import jax
import jax.numpy as jnp
from jax import lax
from jax.experimental import pallas as pl
from jax.experimental.pallas import tpu as pltpu
from jax.experimental.pallas import tpu_sc as plsc

_V = 10000
_VPAD = 10240
_D = 128
_H = 512
_W = 512
_P = _H * _W

_NC = 2
_NS = 16
_NW = _NC * _NS
_CPW = _D // _NW
_CHUNK = 4096
_NCHUNK = _P // _CHUNK
_LANES = 16
_UNROLL = 8


def _transpose_body(x_ref, o_ref):
    o_ref[...] = x_ref[...].T


def _table_transpose(table_pad):
    return pl.pallas_call(
        _transpose_body,
        out_shape=jax.ShapeDtypeStruct((_D, _VPAD), jnp.float32),
    )(table_pad)


def _gather_body(table_t_hbm, idx_hbm, out_hbm, rows_v, idx_v, out_v,
                 sem_idx0, sem_idx1, sem_out0, sem_out1):
    wid = lax.axis_index("s") * _NC + lax.axis_index("c")
    c0 = wid * _CPW
    sem_idx = (sem_idx0, sem_idx1)
    sem_out = (sem_out0, sem_out1)

    def idx_copy(g, b):
        return pltpu.make_async_copy(
            idx_hbm.at[pl.ds(g * _CHUNK, _CHUNK)],
            idx_v.at[pl.ds(b * _CHUNK, _CHUNK)], sem_idx[b])

    def out_copy(g, b, c):
        return pltpu.make_async_copy(
            out_v.at[pl.ds((b * _CPW + c) * _CHUNK, _CHUNK)],
            out_hbm.at[c0 + c, pl.ds(g * _CHUNK, _CHUNK)], sem_out[b])

    idx_copy(0, 0).start()
    pltpu.sync_copy(table_t_hbm.at[pl.ds(c0 * _VPAD, _CPW * _VPAD)], rows_v)

    for g in range(_NCHUNK):
        b = g % 2
        idx_copy(g, b).wait()
        if g + 1 < _NCHUNK:
            idx_copy(g + 1, 1 - b).start()
        if g >= 2:
            for c in range(_CPW):
                out_copy(g - 2, b, c).wait()

        @plsc.parallel_loop(0, _CHUNK // _LANES, unroll=_UNROLL)
        def vec_body(i):
            p = i * _LANES
            idx16 = idx_v[pl.ds(b * _CHUNK + p, _LANES)]
            for c in range(_CPW):
                out_v[pl.ds((b * _CPW + c) * _CHUNK + p, _LANES)] = (
                    plsc.load_gather(rows_v, [idx16 + (c * _VPAD)]))

        for c in range(_CPW):
            out_copy(g, b, c).start()

    for g in (_NCHUNK - 2, _NCHUNK - 1):
        for c in range(_CPW):
            out_copy(g, g % 2, c).wait()


_gather_call = functools.partial(
    pl.kernel,
    out_type=jax.ShapeDtypeStruct((_D, _P), jnp.float32),
    mesh=plsc.VectorSubcoreMesh(core_axis_name="c", subcore_axis_name="s"),
    scratch_types=[
        pltpu.VMEM((_CPW * _VPAD,), jnp.float32),
        pltpu.VMEM((2 * _CHUNK,), jnp.int32),
        pltpu.VMEM((2 * _CPW * _CHUNK,), jnp.float32),
        pltpu.SemaphoreType.DMA,
        pltpu.SemaphoreType.DMA,
        pltpu.SemaphoreType.DMA,
        pltpu.SemaphoreType.DMA,
    ],
    compiler_params=pltpu.CompilerParams(needs_layout_passes=False),
)(_gather_body)


def kernel(graph_nodes, spx_image):
    table_pad = jnp.pad(graph_nodes, ((0, _VPAD - _V), (0, 0)))
    table_t = _table_transpose(table_pad).reshape(_D * _VPAD)
    idx = spx_image.reshape(-1).astype(jnp.int32)
    out = _gather_call(table_t, idx)
    return out.reshape(_D, _H, _W)


# speedup vs baseline: 8.0029x; 1.6612x over previous
import functools

import jax
import jax.numpy as jnp
from jax import lax
from jax.experimental import pallas as pl
from jax.experimental.pallas import tpu as pltpu
from jax.experimental.pallas import tpu_sc as plsc

_V = 10000
_VPAD = 10240
_D = 128
_H = 512
_W = 512
_P = _H * _W

_NC = 2
_NS = 16
_NW = _NC * _NS
_CPW = _D // _NW
_CHUNK = 4096
_NCHUNK = _P // _CHUNK
_LANES = 16
_UNROLL = 8


def _transpose_body(x_ref, o_ref):
    o_ref[...] = x_ref[...].T


def _table_transpose(table_pad):
    return pl.pallas_call(
        _transpose_body,
        out_shape=jax.ShapeDtypeStruct((_D, _VPAD), jnp.float32),
    )(table_pad)


def _gather_body(table_t_hbm, idx_hbm, out_hbm, rows_v, idx_v, out_v,
                 sem_idx0, sem_idx1, sem_out0, sem_out1):
    wid = lax.axis_index("s") * _NC + lax.axis_index("c")
    c0 = wid * _CPW
    sem_idx = (sem_idx0, sem_idx1)
    sem_out = (sem_out0, sem_out1)

    def idx_copy(g, b):
        return pltpu.make_async_copy(
            idx_hbm.at[pl.ds(g * _CHUNK, _CHUNK)],
            idx_v.at[pl.ds(b * _CHUNK, _CHUNK)], sem_idx[b])

    def out_copy(g, b, c):
        return pltpu.make_async_copy(
            out_v.at[b * _CPW + c],
            out_hbm.at[c0 + c].at[pl.ds(g * 8, 8)], sem_out[b])

    idx_copy(0, 0).start()
    pltpu.sync_copy(table_t_hbm.at[pl.ds(c0 * _VPAD, _CPW * _VPAD)], rows_v)

    for g in range(_NCHUNK):
        b = g % 2
        idx_copy(g, b).wait()
        if g + 1 < _NCHUNK:
            idx_copy(g + 1, 1 - b).start()
        if g >= 2:
            for c in range(_CPW):
                out_copy(g - 2, b, c).wait()

        @plsc.parallel_loop(0, _CHUNK // _LANES, unroll=_UNROLL)
        def vec_body(i):
            p = i * _LANES
            h = i // (_W // _LANES)
            w = p % _W
            idx16 = idx_v[pl.ds(b * _CHUNK + p, _LANES)]
            for c in range(_CPW):
                out_v[b * _CPW + c, h, pl.ds(w, _LANES)] = (
                    plsc.load_gather(rows_v, [idx16 + (c * _VPAD)]))

        for c in range(_CPW):
            out_copy(g, b, c).start()

    for g in (_NCHUNK - 2, _NCHUNK - 1):
        for c in range(_CPW):
            out_copy(g, g % 2, c).wait()


_gather_call = functools.partial(
    pl.kernel,
    out_type=jax.ShapeDtypeStruct((_D, _H, _W), jnp.float32),
    mesh=plsc.VectorSubcoreMesh(core_axis_name="c", subcore_axis_name="s"),
    scratch_types=[
        pltpu.VMEM((_CPW * _VPAD,), jnp.float32),
        pltpu.VMEM((2 * _CHUNK,), jnp.int32),
        pltpu.VMEM((2 * _CPW, 8, _W), jnp.float32),
        pltpu.SemaphoreType.DMA,
        pltpu.SemaphoreType.DMA,
        pltpu.SemaphoreType.DMA,
        pltpu.SemaphoreType.DMA,
    ],
    compiler_params=pltpu.CompilerParams(
        needs_layout_passes=False, use_tc_tiling_on_sc=True),
)(_gather_body)


def kernel(graph_nodes, spx_image):
    table_pad = jnp.pad(graph_nodes, ((0, _VPAD - _V), (0, 0)))
    table_t = _table_transpose(table_pad).reshape(_D * _VPAD)
    idx = spx_image.reshape(-1).astype(jnp.int32)
    return _gather_call(table_t, idx)
